# baseline (speedup 1.0000x reference)
import jax
import jax.numpy as jnp
from jax import lax
from jax.experimental import pallas as pl
from jax.experimental.pallas import tpu as pltpu
from jax.experimental.pallas import tpu_sc as plsc

DIM = 1024
EMB_ROWS = 4096
R = 1536
S_ROWS = 2 * R + EMB_ROWS + 8
W = 24


def _make_tc_body(b0):
    def _tc_body(starts_ref, seg_first_ref, n_extra_ref, x_ref, emb_hbm,
                 o_ref, tab_ref, sem):
        b = pl.program_id(0) + b0

        @pl.when(pl.program_id(0) == 0)
        def _():
            pltpu.make_async_copy(
                emb_hbm.at[pl.ds(0, EMB_ROWS), :],
                tab_ref.at[pl.ds(R, EMB_ROWS), :],
                sem,
            ).start()
            pltpu.make_async_copy(
                emb_hbm.at[pl.ds(0, EMB_ROWS), :],
                tab_ref.at[pl.ds(R, EMB_ROWS), :],
                sem,
            ).wait()

        base = b * R
        s0 = seg_first_ref[b]

        off0 = base - starts_ref[s0] + R
        q0 = pl.multiple_of((off0 // 8) * 8, 8)

        def store_piece0(k):
            def f():
                big = tab_ref[pl.ds(q0, R + 8), :]
                o_ref[:, :] = big[k:k + R, :] + x_ref[:, :]
            return f

        lax.switch(off0 - q0, [store_piece0(k) for k in range(8)])

        riota = lax.broadcasted_iota(jnp.int32, (R, 1), 0)

        def piece(j, carry):
            st = starts_ref[s0 + j]
            boundary = st - base
            offj = base - st + R
            qj = pl.multiple_of((offj // 8) * 8, 8)

            def store_piecej(k):
                def f():
                    big = tab_ref[pl.ds(qj, R + 8), :]
                    o_ref[:, :] = jnp.where(
                        riota >= boundary,
                        big[k:k + R, :] + x_ref[:, :],
                        o_ref[:, :])
                return f

            lax.switch(offj - qj, [store_piecej(k) for k in range(8)])
            return carry

        lax.fori_loop(1, n_extra_ref[b] + 1, piece, 0)

    return _tc_body


def _tc_call(x, emb, starts, seg_first, n_extra, b0=0, carry=None,
             interpret=False):
    total = x.shape[0]
    nb = (total + R - 1) // R
    inputs = [starts, seg_first, n_extra, x, emb]
    in_specs = [
        pl.BlockSpec((R, DIM), lambda b, *_: (b + b0, 0)),
        pl.BlockSpec(memory_space=pltpu.MemorySpace.HBM),
    ]
    aliases = {}
    if carry is not None:
        inputs.append(carry)
        in_specs.append(pl.BlockSpec(memory_space=pltpu.MemorySpace.HBM))
        aliases = {5: 0}
    grid_spec = pltpu.PrefetchScalarGridSpec(
        num_scalar_prefetch=3,
        grid=(nb - b0,),
        in_specs=in_specs,
        out_specs=pl.BlockSpec((R, DIM), lambda b, *_: (b + b0, 0)),
        scratch_shapes=[
            pltpu.VMEM((S_ROWS, DIM), jnp.float32),
            pltpu.SemaphoreType.DMA,
        ],
    )
    body = _make_tc_body(b0)
    if carry is not None:
        def body_w(starts_ref, seg_first_ref, n_extra_ref, x_ref, emb_hbm,
                   carry_hbm, o_ref, tab_ref, sem, _b=body):
            _b(starts_ref, seg_first_ref, n_extra_ref, x_ref, emb_hbm,
               o_ref, tab_ref, sem)
    else:
        body_w = body
    return pl.pallas_call(
        body_w,
        grid_spec=grid_spec,
        out_shape=jax.ShapeDtypeStruct((total, DIM), jnp.float32),
        compiler_params=pltpu.CompilerParams(
            dimension_semantics=("arbitrary",)),
        input_output_aliases=aliases,
        interpret=interpret,
    )(*inputs)


def _sc_call(x, emb, idx2):
    total = x.shape[0]
    nwin = idx2.shape[0]
    mesh = plsc.VectorSubcoreMesh(
        core_axis_name="core", subcore_axis_name="subcore")

    @pl.kernel(
        out_type=jax.ShapeDtypeStruct((total, DIM), jnp.float32),
        mesh=mesh,
    )
    def k(x_hbm, emb_hbm, i_hbm, o_hbm):
        def body(i_vmem, x_vmem, o_vmem):
            pltpu.sync_copy(emb_hbm.at[i_vmem.at[0, pl.ds(0, W)]], o_vmem)

            @pl.loop(0, W)
            def _(r):
                @pl.loop(0, DIM, step=16)
                def _(c):
                    o_vmem.at[r, pl.ds(c, 16)][...] = (
                        o_vmem.at[r, pl.ds(c, 16)][...]
                        + x_vmem.at[r, pl.ds(c, 16)][...])

        pltpu.emit_pipeline(
            body,
            grid=(nwin,),
            in_specs=[
                pl.BlockSpec((1, 128), index_map=lambda i: (i, 0)),
                pl.BlockSpec((W, DIM), index_map=lambda i: (i, 0)),
            ],
            out_specs=[pl.BlockSpec((W, DIM), index_map=lambda i: (i, 0))],
            core_axis_name=("core", "subcore"),
            dimension_semantics=(pltpu.PARALLEL,),
        )(i_hbm, x_hbm, o_hbm)

    return k(x, emb, idx2)


def kernel(x, emb, sizes):
    total = x.shape[0]
    nb = (total + R - 1) // R
    sizes = sizes.astype(jnp.int32)
    csum = jnp.cumsum(sizes)
    starts = (csum - sizes).astype(jnp.int32)
    bstart = jnp.arange(nb, dtype=jnp.int32) * R
    seg_first = jnp.searchsorted(csum, bstart, side="right").astype(jnp.int32)
    last_row = jnp.minimum(bstart + (R - 1), total - 1)
    seg_last = jnp.searchsorted(csum, last_row, side="right").astype(jnp.int32)
    n_extra = seg_last - seg_first

    b0 = total // R
    nwin = (b0 * R) // W
    if nwin == 0:
        return _tc_call(x, emb, starts, seg_first, n_extra)

    ii = jnp.arange(nwin * W, dtype=jnp.int32)
    seg = jnp.searchsorted(csum, ii, side="right")
    relpos = (ii - jnp.take(starts, seg)).astype(jnp.int32)
    idx2 = jnp.pad(relpos.reshape(nwin, W), ((0, 0), (0, 128 - W)))

    out_sc = _sc_call(x, emb, idx2)
    if b0 == nb:
        return out_sc
    return _tc_call(x, emb, starts, seg_first, n_extra, b0=b0, carry=out_sc)


# speedup vs baseline: 5.5736x; 5.5736x over previous
import jax
import jax.numpy as jnp
from jax import lax
from jax.experimental import pallas as pl
from jax.experimental.pallas import tpu as pltpu

DIM = 1024
EMB_ROWS = 4096
R = 1536
S_ROWS = 2 * R + EMB_ROWS + 8


def _make_tc_body(nb):
    return lambda *args: _tc_body(nb, *args)


def _tc_body(nb, starts_ref, seg_first_ref, n_extra_ref, x_ref, emb_hbm,
             o_ref, tab_ref, sem):
    b = pl.program_id(0)
    rest = EMB_ROWS - (R + 8)

    @pl.when(b == 0)
    def _():
        head = pltpu.make_async_copy(
            emb_hbm.at[pl.ds(0, R + 8), :],
            tab_ref.at[pl.ds(R, R + 8), :],
            sem.at[0],
        )
        head.start()
        if nb > 1:
            pltpu.make_async_copy(
                emb_hbm.at[pl.ds(R + 8, rest), :],
                tab_ref.at[pl.ds(2 * R + 8, rest), :],
                sem.at[1],
            ).start()
        head.wait()

    if nb > 1:
        @pl.when(b == 1)
        def _():
            pltpu.make_async_copy(
                emb_hbm.at[pl.ds(R + 8, rest), :],
                tab_ref.at[pl.ds(2 * R + 8, rest), :],
                sem.at[1],
            ).wait()

    base = b * R
    s0 = seg_first_ref[b]

    off0 = base - starts_ref[s0] + R
    q0 = pl.multiple_of((off0 // 8) * 8, 8)

    def store_piece0(k):
        def f():
            big = tab_ref[pl.ds(q0, R + 8), :]
            o_ref[:, :] = big[k:k + R, :] + x_ref[:, :]
        return f

    lax.switch(off0 - q0, [store_piece0(k) for k in range(8)])

    riota = lax.broadcasted_iota(jnp.int32, (R, 1), 0)

    def piece(j, carry):
        st = starts_ref[s0 + j]
        boundary = st - base
        offj = base - st + R
        qj = pl.multiple_of((offj // 8) * 8, 8)

        def store_piecej(k):
            def f():
                big = tab_ref[pl.ds(qj, R + 8), :]
                o_ref[:, :] = jnp.where(
                    riota >= boundary,
                    big[k:k + R, :] + x_ref[:, :],
                    o_ref[:, :])
            return f

        lax.switch(offj - qj, [store_piecej(k) for k in range(8)])
        return carry

    lax.fori_loop(1, n_extra_ref[b] + 1, piece, 0)


def _tc_call(x, emb, starts, seg_first, n_extra, interpret=False):
    total = x.shape[0]
    nb = (total + R - 1) // R
    grid_spec = pltpu.PrefetchScalarGridSpec(
        num_scalar_prefetch=3,
        grid=(nb,),
        in_specs=[
            pl.BlockSpec((R, DIM), lambda b, *_: (b, 0)),
            pl.BlockSpec(memory_space=pltpu.MemorySpace.HBM),
        ],
        out_specs=pl.BlockSpec((R, DIM), lambda b, *_: (b, 0)),
        scratch_shapes=[
            pltpu.VMEM((S_ROWS, DIM), jnp.float32),
            pltpu.SemaphoreType.DMA((2,)),
        ],
    )
    return pl.pallas_call(
        _make_tc_body(nb),
        grid_spec=grid_spec,
        out_shape=jax.ShapeDtypeStruct((total, DIM), jnp.float32),
        compiler_params=pltpu.CompilerParams(
            dimension_semantics=("arbitrary",)),
        interpret=interpret,
    )(starts, seg_first, n_extra, x, emb)


def kernel(x, emb, sizes):
    total = x.shape[0]
    nb = (total + R - 1) // R
    sizes = sizes.astype(jnp.int32)
    csum = jnp.cumsum(sizes)
    starts = (csum - sizes).astype(jnp.int32)
    bstart = jnp.arange(nb, dtype=jnp.int32) * R
    seg_first = jnp.searchsorted(csum, bstart, side="right").astype(jnp.int32)
    last_row = jnp.minimum(bstart + (R - 1), total - 1)
    seg_last = jnp.searchsorted(csum, last_row, side="right").astype(jnp.int32)
    n_extra = seg_last - seg_first
    return _tc_call(x, emb, starts, seg_first, n_extra)


# speedup vs baseline: 5.8100x; 1.0424x over previous
import jax
import jax.numpy as jnp
from jax import lax
from jax.experimental import pallas as pl
from jax.experimental.pallas import tpu as pltpu

DIM = 1024
EMB_ROWS = 4096
R = 1664
S_ROWS = 2 * R + EMB_ROWS + 8


def _make_tc_body(nb):
    return lambda *args: _tc_body(nb, *args)


def _tc_body(nb, starts_ref, seg_first_ref, n_extra_ref, x_ref, emb_hbm,
             o_ref, tab_ref, sem):
    b = pl.program_id(0)
    rest = EMB_ROWS - (R + 8)

    @pl.when(b == 0)
    def _():
        head = pltpu.make_async_copy(
            emb_hbm.at[pl.ds(0, R + 8), :],
            tab_ref.at[pl.ds(R, R + 8), :],
            sem.at[0],
        )
        head.start()
        if nb > 1:
            pltpu.make_async_copy(
                emb_hbm.at[pl.ds(R + 8, rest), :],
                tab_ref.at[pl.ds(2 * R + 8, rest), :],
                sem.at[1],
            ).start()
        head.wait()

    if nb > 1:
        @pl.when(b == 1)
        def _():
            pltpu.make_async_copy(
                emb_hbm.at[pl.ds(R + 8, rest), :],
                tab_ref.at[pl.ds(2 * R + 8, rest), :],
                sem.at[1],
            ).wait()

    base = b * R
    s0 = seg_first_ref[b]

    off0 = base - starts_ref[s0] + R
    q0 = pl.multiple_of((off0 // 8) * 8, 8)

    def store_piece0(k):
        def f():
            big = tab_ref[pl.ds(q0, R + 8), :]
            o_ref[:, :] = big[k:k + R, :] + x_ref[:, :]
        return f

    lax.switch(off0 - q0, [store_piece0(k) for k in range(8)])

    riota = lax.broadcasted_iota(jnp.int32, (R, 1), 0)

    def piece(j, carry):
        st = starts_ref[s0 + j]
        boundary = st - base
        offj = base - st + R
        qj = pl.multiple_of((offj // 8) * 8, 8)

        def store_piecej(k):
            def f():
                big = tab_ref[pl.ds(qj, R + 8), :]
                o_ref[:, :] = jnp.where(
                    riota >= boundary,
                    big[k:k + R, :] + x_ref[:, :],
                    o_ref[:, :])
            return f

        lax.switch(offj - qj, [store_piecej(k) for k in range(8)])
        return carry

    lax.fori_loop(1, n_extra_ref[b] + 1, piece, 0)


def _tc_call(x, emb, starts, seg_first, n_extra, interpret=False):
    total = x.shape[0]
    nb = (total + R - 1) // R
    grid_spec = pltpu.PrefetchScalarGridSpec(
        num_scalar_prefetch=3,
        grid=(nb,),
        in_specs=[
            pl.BlockSpec((R, DIM), lambda b, *_: (b, 0)),
            pl.BlockSpec(memory_space=pltpu.MemorySpace.HBM),
        ],
        out_specs=pl.BlockSpec((R, DIM), lambda b, *_: (b, 0)),
        scratch_shapes=[
            pltpu.VMEM((S_ROWS, DIM), jnp.float32),
            pltpu.SemaphoreType.DMA((2,)),
        ],
    )
    return pl.pallas_call(
        _make_tc_body(nb),
        grid_spec=grid_spec,
        out_shape=jax.ShapeDtypeStruct((total, DIM), jnp.float32),
        compiler_params=pltpu.CompilerParams(
            dimension_semantics=("arbitrary",)),
        interpret=interpret,
    )(starts, seg_first, n_extra, x, emb)


def kernel(x, emb, sizes):
    total = x.shape[0]
    nb = (total + R - 1) // R
    sizes = sizes.astype(jnp.int32)
    csum = jnp.cumsum(sizes)
    starts = (csum - sizes).astype(jnp.int32)
    bstart = jnp.arange(nb, dtype=jnp.int32) * R
    seg_first = jnp.searchsorted(csum, bstart, side="right").astype(jnp.int32)
    last_row = jnp.minimum(bstart + (R - 1), total - 1)
    seg_last = jnp.searchsorted(csum, last_row, side="right").astype(jnp.int32)
    n_extra = seg_last - seg_first
    return _tc_call(x, emb, starts, seg_first, n_extra)
